# baseline (speedup 1.0000x reference)
import functools

import jax
import jax.numpy as jnp
from jax import lax
from jax.experimental import pallas as pl
from jax.experimental.pallas import tpu as pltpu
from jax.experimental.pallas import tpu_sc as plsc

N = 10000
E = 320000
D = 128
G = 16

NC = 2
NS = 16
NW = NC * NS
EPW = E // NW
CHUNK = 125
NCHUNK = EPW // CHUNK
ROWS_PER_TILE = N // NS

_mesh = plsc.VectorSubcoreMesh(core_axis_name="c", subcore_axis_name="s")


def _deg_body(dst_hbm, out_hbm, dst_v, ones_v, zbuf, acc):
    cid = lax.axis_index("c")
    sid = lax.axis_index("s")
    w = cid * NS + sid

    def zinit(i, _):
        zbuf[pl.ds(i * 16, 16)] = jnp.zeros((16,), jnp.float32)
        return 0

    lax.fori_loop(0, N // 16, zinit, 0)

    @pl.when(sid == 0)
    def _():
        pltpu.sync_copy(zbuf, acc)

    def oinit(i, _):
        ones_v[pl.ds(i * 16, 16)] = jnp.ones((16,), jnp.float32)
        return 0

    lax.fori_loop(0, CHUNK // 16 + 1, oinit, 0)
    plsc.subcore_barrier()

    pltpu.sync_copy(dst_hbm.at[w], dst_v)

    def body(j, _):
        pltpu.sync_copy(ones_v.at[pl.ds(0, CHUNK)], acc.at[dst_v.at[j]],
                        add=True)
        return 0

    lax.fori_loop(0, NCHUNK, body, 0)
    plsc.subcore_barrier()

    @pl.when(sid == 0)
    def _():
        pltpu.sync_copy(acc, out_hbm.at[cid])


_deg_call = pl.kernel(
    _deg_body,
    out_type=jax.ShapeDtypeStruct((NC, N), jnp.float32),
    mesh=_mesh,
    compiler_params=pltpu.CompilerParams(use_tc_tiling_on_sc=False),
    scratch_types=[
        pltpu.VMEM((NCHUNK, CHUNK), jnp.int32),
        pltpu.VMEM((CHUNK + 16 - CHUNK % 16,), jnp.float32),
        pltpu.VMEM((N,), jnp.float32),
        pltpu.VMEM_SHARED((N,), jnp.float32),
    ],
)


CD = D // NC
EPW2 = E // NW
ECH = 100
NCH2 = EPW2 // ECH
NSTG = 4
HSTG = NCH2 // NSTG


def _edge_body(hp_hbm, src_hbm, dst_hbm, zeros_hbm, out_hbm,
               src_v, dst_v, buf0, buf1, buf2, acc, sem0, sem1, sem2):
    cid = lax.axis_index("c")
    sid = lax.axis_index("s")
    w = cid * NS + sid

    base = sid * ROWS_PER_TILE
    pltpu.sync_copy(zeros_hbm, acc.at[pl.ds(base, ROWS_PER_TILE)])
    plsc.subcore_barrier()

    bufs = ((buf0, sem0), (buf1, sem1), (buf2, sem2))
    NB = len(bufs)

    for q in range(NSTG):
        pltpu.sync_copy(src_hbm.at[w].at[pl.ds(q * HSTG, HSTG)], src_v)
        pltpu.sync_copy(dst_hbm.at[w].at[pl.ds(q * HSTG, HSTG)], dst_v)

        for b, (buf, sem) in enumerate(bufs):
            pltpu.async_copy(hp_hbm.at[src_v.at[b]], buf, sem)

        def body(jj, _):
            for b, (buf, sem) in enumerate(bufs):
                j = jj * NB + b
                pltpu.make_async_copy(hp_hbm.at[src_v.at[j]], buf,
                                      sem).wait()
                pltpu.sync_copy(buf, acc.at[dst_v.at[j]], add=True)

                @pl.when(j + NB < HSTG)
                def _():
                    pltpu.async_copy(hp_hbm.at[src_v.at[j + NB]], buf, sem)

            return 0

        lax.fori_loop(0, HSTG // NB, body, 0)

        j = HSTG - 1
        pltpu.make_async_copy(hp_hbm.at[src_v.at[j]], buf0, sem0).wait()
        pltpu.sync_copy(buf0, acc.at[dst_v.at[j]], add=True)

    plsc.subcore_barrier()

    pltpu.sync_copy(acc.at[pl.ds(base, ROWS_PER_TILE)],
                    out_hbm.at[cid].at[pl.ds(base, ROWS_PER_TILE)])


_edge_call = pl.kernel(
    _edge_body,
    out_type=jax.ShapeDtypeStruct((NC, N, D), jnp.float32),
    mesh=_mesh,
    compiler_params=pltpu.CompilerParams(use_tc_tiling_on_sc=False),
    scratch_types=[
        pltpu.VMEM((HSTG, ECH), jnp.int32),
        pltpu.VMEM((HSTG, ECH), jnp.int32),
        pltpu.VMEM((ECH, D), jnp.float32),
        pltpu.VMEM((ECH, D), jnp.float32),
        pltpu.VMEM((ECH, D), jnp.float32),
        pltpu.VMEM_SHARED((N, D), jnp.float32),
        pltpu.SemaphoreType.DMA,
        pltpu.SemaphoreType.DMA,
        pltpu.SemaphoreType.DMA,
    ],
)


def _k0_body(degp_ref, x_ref, w1_ref, dinv_ref, hp_ref):
    deg = jnp.sum(degp_ref[...], axis=0) + 1.0
    dinv = lax.rsqrt(deg)
    dinv_ref[...] = dinv
    h = jnp.dot(x_ref[...], w1_ref[...], preferred_element_type=jnp.float32)
    hp_ref[...] = h * dinv[:, None]


def _k0(deg_parts, x, W1):
    return pl.pallas_call(
        _k0_body,
        out_shape=[
            jax.ShapeDtypeStruct((N,), jnp.float32),
            jax.ShapeDtypeStruct((N, D), jnp.float32),
        ],
    )(deg_parts, x, W1)


def _fuse_mid_body(pacc_ref, hp_ref, dinv_ref, b_ref, g_ref, be_ref,
                   w_ref, out_ref):
    dinv = dinv_ref[...]
    s = pacc_ref[0] + pacc_ref[1] + hp_ref[...]
    agg = s * dinv[:, None] + b_ref[...][None, :]
    mean = jnp.mean(agg, axis=0)
    var = jnp.mean(agg * agg, axis=0) - mean * mean
    xn = (agg - mean[None, :]) * lax.rsqrt(var + 1e-5)[None, :]
    h = jnp.maximum(xn * g_ref[...][None, :] + be_ref[...][None, :], 0.0)
    hw = jnp.dot(h, w_ref[...], preferred_element_type=jnp.float32)
    out_ref[...] = hw * dinv[:, None]


def _fuse_mid(pacc, hp, dinv, b, g, be, W):
    return pl.pallas_call(
        _fuse_mid_body,
        out_shape=jax.ShapeDtypeStruct((N, D), jnp.float32),
    )(pacc, hp, dinv, b, g, be, W)


def _fuse_tail_body(pacc_ref, hp_ref, dinv_ref, b_ref, gw1_ref, gb1_ref,
                    gw2_ref, gb2_ref, batch_ref, mw1_ref, mb1_ref, mw2_ref,
                    mb2_ref, out_ref):
    s = pacc_ref[0] + pacc_ref[1] + hp_ref[...]
    h3 = s * dinv_ref[...][:, None] + b_ref[...][None, :]
    gmid = jnp.maximum(
        jnp.dot(h3, gw1_ref[...], preferred_element_type=jnp.float32)
        + gb1_ref[...][None, :], 0.0)
    gate = (jnp.dot(gmid, gw2_ref[...], preferred_element_type=jnp.float32)
            + gb2_ref[...][None, :])[:, 0]
    oh = batch_ref[...][:, None] == lax.broadcasted_iota(jnp.int32, (1, G), 1)
    mx = jnp.max(jnp.where(oh, gate[:, None], -jnp.inf), axis=0)
    mx = jnp.where(jnp.isfinite(mx), mx, 0.0)
    ohf = oh.astype(jnp.float32)
    mxb = jnp.dot(ohf, mx[:, None], preferred_element_type=jnp.float32)[:, 0]
    e = jnp.exp(gate - mxb)
    den = jnp.sum(ohf * e[:, None], axis=0)
    ew = lax.dot_general(ohf, h3 * e[:, None], (((0,), (0,)), ((), ())),
                         preferred_element_type=jnp.float32)
    pooled = ew / (den + 1e-16)[:, None]
    m1 = jnp.maximum(
        jnp.dot(pooled, mw1_ref[...], preferred_element_type=jnp.float32)
        + mb1_ref[...][None, :], 0.0)
    out_ref[...] = (jnp.dot(m1, mw2_ref[...],
                            preferred_element_type=jnp.float32)
                    + mb2_ref[...][None, :])


def _fuse_tail(pacc, hp, dinv, b, gW1, gb1, gW2, gb2, batch,
               mW1, mb1, mW2, mb2):
    return pl.pallas_call(
        _fuse_tail_body,
        out_shape=jax.ShapeDtypeStruct((G, D), jnp.float32),
    )(pacc, hp, dinv, b, gW1, gb1, gW2, gb2, batch, mW1, mb1, mW2, mb2)


def kernel(x, edge_index, batch, W1, b1, W2, b2, W3, b3, g1, be1, g2, be2,
           gW1, gb1, gW2, gb2, mW1, mb1, mW2, mb2):
    src3 = edge_index[0].reshape(NW, NCH2, ECH)
    dst3 = edge_index[1].reshape(NW, NCH2, ECH)
    dst3d = edge_index[1].reshape(NW, NCHUNK, CHUNK)

    deg_parts = _deg_call(dst3d)
    dinv, hp1 = _k0(deg_parts, x, W1)

    zrows = jnp.zeros((ROWS_PER_TILE, D), jnp.float32)
    pacc1 = _edge_call(hp1, src3, dst3, zrows)
    hp2 = _fuse_mid(pacc1, hp1, dinv, b1, g1, be1, W2)

    pacc2 = _edge_call(hp2, src3, dst3, zrows)
    hp3 = _fuse_mid(pacc2, hp2, dinv, b2, g2, be2, W3)

    pacc3 = _edge_call(hp3, src3, dst3, zrows)
    return _fuse_tail(pacc3, hp3, dinv, b3, gW1, gb1, gW2, gb2, batch,
                      mW1, mb1, mW2, mb2)


# speedup vs baseline: 1.0297x; 1.0297x over previous
import functools

import jax
import jax.numpy as jnp
from jax import lax
from jax.experimental import pallas as pl
from jax.experimental.pallas import tpu as pltpu
from jax.experimental.pallas import tpu_sc as plsc

N = 10000
E = 320000
D = 128
G = 16

NC = 2
NS = 16
NW = NC * NS
EPW = E // NW
CHUNK = 125
NCHUNK = EPW // CHUNK
ROWS_PER_TILE = N // NS

_mesh = plsc.VectorSubcoreMesh(core_axis_name="c", subcore_axis_name="s")


def _deg_body(dst_hbm, out_hbm, dst_v, ones_v, zbuf, acc):
    cid = lax.axis_index("c")
    sid = lax.axis_index("s")
    w = cid * NS + sid

    def zinit(i, _):
        zbuf[pl.ds(i * 16, 16)] = jnp.zeros((16,), jnp.float32)
        return 0

    lax.fori_loop(0, N // 16, zinit, 0)

    @pl.when(sid == 0)
    def _():
        pltpu.sync_copy(zbuf, acc)

    def oinit(i, _):
        ones_v[pl.ds(i * 16, 16)] = jnp.ones((16,), jnp.float32)
        return 0

    lax.fori_loop(0, CHUNK // 16 + 1, oinit, 0)
    plsc.subcore_barrier()

    pltpu.sync_copy(dst_hbm.at[w], dst_v)

    def body(j, _):
        pltpu.sync_copy(ones_v.at[pl.ds(0, CHUNK)], acc.at[dst_v.at[j]],
                        add=True)
        return 0

    lax.fori_loop(0, NCHUNK, body, 0)
    plsc.subcore_barrier()

    @pl.when(sid == 0)
    def _():
        pltpu.sync_copy(acc, out_hbm.at[cid])


_deg_call = pl.kernel(
    _deg_body,
    out_type=jax.ShapeDtypeStruct((NC, N), jnp.float32),
    mesh=_mesh,
    compiler_params=pltpu.CompilerParams(use_tc_tiling_on_sc=False),
    scratch_types=[
        pltpu.VMEM((NCHUNK, CHUNK), jnp.int32),
        pltpu.VMEM((CHUNK + 16 - CHUNK % 16,), jnp.float32),
        pltpu.VMEM((N,), jnp.float32),
        pltpu.VMEM_SHARED((N,), jnp.float32),
    ],
)


CD = D // NC
EPW2 = E // NW
ECH = 100
NCH2 = EPW2 // ECH
NSTG = 4
HSTG = NCH2 // NSTG


def _edge_body(hp_hbm, src_hbm, dst_hbm, out_hbm,
               src_v, dst_v, buf0, buf1, buf2, acc, sem0, sem1, sem2):
    cid = lax.axis_index("c")
    sid = lax.axis_index("s")
    w = cid * NS + sid

    def zrow(r, _):
        for c in range(D // 16):
            buf0[r, pl.ds(c * 16, 16)] = jnp.zeros((16,), jnp.float32)
        return 0

    lax.fori_loop(0, ECH, zrow, 0)
    base = sid * ROWS_PER_TILE
    for k in range(6):
        pltpu.sync_copy(buf0, acc.at[pl.ds(base + k * ECH, ECH)])
    pltpu.sync_copy(buf0.at[pl.ds(0, 25)], acc.at[pl.ds(base + 600, 25)])
    plsc.subcore_barrier()

    bufs = ((buf0, sem0), (buf1, sem1), (buf2, sem2))
    NB = len(bufs)

    for q in range(NSTG):
        pltpu.sync_copy(src_hbm.at[w].at[pl.ds(q * HSTG, HSTG)], src_v)
        pltpu.sync_copy(dst_hbm.at[w].at[pl.ds(q * HSTG, HSTG)], dst_v)

        for b, (buf, sem) in enumerate(bufs):
            pltpu.async_copy(hp_hbm.at[src_v.at[b]], buf, sem)

        def body(jj, _):
            for b, (buf, sem) in enumerate(bufs):
                j = jj * NB + b
                pltpu.make_async_copy(hp_hbm.at[src_v.at[j]], buf,
                                      sem).wait()
                pltpu.sync_copy(buf, acc.at[dst_v.at[j]], add=True)

                @pl.when(j + NB < HSTG)
                def _():
                    pltpu.async_copy(hp_hbm.at[src_v.at[j + NB]], buf, sem)

            return 0

        lax.fori_loop(0, HSTG // NB, body, 0)

        j = HSTG - 1
        pltpu.make_async_copy(hp_hbm.at[src_v.at[j]], buf0, sem0).wait()
        pltpu.sync_copy(buf0, acc.at[dst_v.at[j]], add=True)

    plsc.subcore_barrier()

    ocid = out_hbm.at[cid]
    for k in range(6):
        r0 = base + k * ECH
        pltpu.sync_copy(acc.at[pl.ds(r0, ECH)], buf0)
        pltpu.sync_copy(buf0, ocid.at[pl.ds(r0, ECH)])
    r0 = base + 600
    pltpu.sync_copy(acc.at[pl.ds(r0, 25)], buf0.at[pl.ds(0, 25)])
    pltpu.sync_copy(buf0.at[pl.ds(0, 25)], ocid.at[pl.ds(r0, 25)])


_edge_call = pl.kernel(
    _edge_body,
    out_type=jax.ShapeDtypeStruct((NC, N, D), jnp.float32),
    mesh=_mesh,
    compiler_params=pltpu.CompilerParams(use_tc_tiling_on_sc=False),
    scratch_types=[
        pltpu.VMEM((HSTG, ECH), jnp.int32),
        pltpu.VMEM((HSTG, ECH), jnp.int32),
        pltpu.VMEM((ECH, D), jnp.float32),
        pltpu.VMEM((ECH, D), jnp.float32),
        pltpu.VMEM((ECH, D), jnp.float32),
        pltpu.VMEM_SHARED((N, D), jnp.float32),
        pltpu.SemaphoreType.DMA,
        pltpu.SemaphoreType.DMA,
        pltpu.SemaphoreType.DMA,
    ],
)


def _mm1_body(x_ref, w1_ref, h_ref):
    h_ref[...] = jnp.dot(x_ref[...], w1_ref[...],
                         preferred_element_type=jnp.float32)


def _mm1(x, W1):
    return pl.pallas_call(
        _mm1_body,
        out_shape=jax.ShapeDtypeStruct((N, D), jnp.float32),
    )(x, W1)


def _k0_body(degp_ref, h_ref, dinv_ref, hp_ref):
    deg = jnp.sum(degp_ref[...], axis=0) + 1.0
    dinv = lax.rsqrt(deg)
    dinv_ref[...] = dinv
    hp_ref[...] = h_ref[...] * dinv[:, None]


def _k0(deg_parts, h1):
    return pl.pallas_call(
        _k0_body,
        out_shape=[
            jax.ShapeDtypeStruct((N,), jnp.float32),
            jax.ShapeDtypeStruct((N, D), jnp.float32),
        ],
    )(deg_parts, h1)


def _fuse_mid_body(pacc_ref, hp_ref, dinv_ref, b_ref, g_ref, be_ref,
                   w_ref, out_ref):
    dinv = dinv_ref[...]
    s = pacc_ref[0] + pacc_ref[1] + hp_ref[...]
    agg = s * dinv[:, None] + b_ref[...][None, :]
    mean = jnp.mean(agg, axis=0)
    var = jnp.mean(agg * agg, axis=0) - mean * mean
    xn = (agg - mean[None, :]) * lax.rsqrt(var + 1e-5)[None, :]
    h = jnp.maximum(xn * g_ref[...][None, :] + be_ref[...][None, :], 0.0)
    hw = jnp.dot(h, w_ref[...], preferred_element_type=jnp.float32)
    out_ref[...] = hw * dinv[:, None]


def _fuse_mid(pacc, hp, dinv, b, g, be, W):
    return pl.pallas_call(
        _fuse_mid_body,
        out_shape=jax.ShapeDtypeStruct((N, D), jnp.float32),
    )(pacc, hp, dinv, b, g, be, W)


def _fuse_tail_body(pacc_ref, hp_ref, dinv_ref, b_ref, gw1_ref, gb1_ref,
                    gw2_ref, gb2_ref, batch_ref, mw1_ref, mb1_ref, mw2_ref,
                    mb2_ref, out_ref):
    s = pacc_ref[0] + pacc_ref[1] + hp_ref[...]
    h3 = s * dinv_ref[...][:, None] + b_ref[...][None, :]
    gmid = jnp.maximum(
        jnp.dot(h3, gw1_ref[...], preferred_element_type=jnp.float32)
        + gb1_ref[...][None, :], 0.0)
    gate = (jnp.dot(gmid, gw2_ref[...], preferred_element_type=jnp.float32)
            + gb2_ref[...][None, :])[:, 0]
    oh = batch_ref[...][:, None] == lax.broadcasted_iota(jnp.int32, (1, G), 1)
    mx = jnp.max(jnp.where(oh, gate[:, None], -jnp.inf), axis=0)
    mx = jnp.where(jnp.isfinite(mx), mx, 0.0)
    ohf = oh.astype(jnp.float32)
    mxb = jnp.dot(ohf, mx[:, None], preferred_element_type=jnp.float32)[:, 0]
    e = jnp.exp(gate - mxb)
    den = jnp.sum(ohf * e[:, None], axis=0)
    ew = lax.dot_general(ohf, h3 * e[:, None], (((0,), (0,)), ((), ())),
                         preferred_element_type=jnp.float32)
    pooled = ew / (den + 1e-16)[:, None]
    m1 = jnp.maximum(
        jnp.dot(pooled, mw1_ref[...], preferred_element_type=jnp.float32)
        + mb1_ref[...][None, :], 0.0)
    out_ref[...] = (jnp.dot(m1, mw2_ref[...],
                            preferred_element_type=jnp.float32)
                    + mb2_ref[...][None, :])


def _fuse_tail(pacc, hp, dinv, b, gW1, gb1, gW2, gb2, batch,
               mW1, mb1, mW2, mb2):
    return pl.pallas_call(
        _fuse_tail_body,
        out_shape=jax.ShapeDtypeStruct((G, D), jnp.float32),
    )(pacc, hp, dinv, b, gW1, gb1, gW2, gb2, batch, mW1, mb1, mW2, mb2)


def kernel(x, edge_index, batch, W1, b1, W2, b2, W3, b3, g1, be1, g2, be2,
           gW1, gb1, gW2, gb2, mW1, mb1, mW2, mb2):
    src3 = edge_index[0].reshape(NW, NCH2, ECH)
    dst3 = edge_index[1].reshape(NW, NCH2, ECH)
    dst3d = edge_index[1].reshape(NW, NCHUNK, CHUNK)

    deg_parts = _deg_call(dst3d)
    h1 = _mm1(x, W1)
    dinv, hp1 = _k0(deg_parts, h1)

    pacc1 = _edge_call(hp1, src3, dst3)
    hp2 = _fuse_mid(pacc1, hp1, dinv, b1, g1, be1, W2)

    pacc2 = _edge_call(hp2, src3, dst3)
    hp3 = _fuse_mid(pacc2, hp2, dinv, b2, g2, be2, W3)

    pacc3 = _edge_call(hp3, src3, dst3)
    return _fuse_tail(pacc3, hp3, dinv, b3, gW1, gb1, gW2, gb2, batch,
                      mW1, mb1, mW2, mb2)


# speedup vs baseline: 1.0302x; 1.0005x over previous
import functools

import jax
import jax.numpy as jnp
from jax import lax
from jax.experimental import pallas as pl
from jax.experimental.pallas import tpu as pltpu
from jax.experimental.pallas import tpu_sc as plsc

N = 10000
E = 320000
D = 128
G = 16

NC = 2
NS = 16
NW = NC * NS
EPW = E // NW
CHUNK = 125
NCHUNK = EPW // CHUNK
ROWS_PER_TILE = N // NS

_mesh = plsc.VectorSubcoreMesh(core_axis_name="c", subcore_axis_name="s")


def _deg_body(dst_hbm, out_hbm, dst_v, ones_v, zbuf, acc):
    cid = lax.axis_index("c")
    sid = lax.axis_index("s")
    w = cid * NS + sid

    def zinit(i, _):
        zbuf[pl.ds(i * 16, 16)] = jnp.zeros((16,), jnp.float32)
        return 0

    lax.fori_loop(0, N // 16, zinit, 0)

    @pl.when(sid == 0)
    def _():
        pltpu.sync_copy(zbuf, acc)

    def oinit(i, _):
        ones_v[pl.ds(i * 16, 16)] = jnp.ones((16,), jnp.float32)
        return 0

    lax.fori_loop(0, CHUNK // 16 + 1, oinit, 0)
    plsc.subcore_barrier()

    pltpu.sync_copy(dst_hbm.at[w], dst_v)

    def body(j, _):
        pltpu.sync_copy(ones_v.at[pl.ds(0, CHUNK)], acc.at[dst_v.at[j]],
                        add=True)
        return 0

    lax.fori_loop(0, NCHUNK, body, 0)
    plsc.subcore_barrier()

    @pl.when(sid == 0)
    def _():
        pltpu.sync_copy(acc, out_hbm.at[cid])


_deg_call = pl.kernel(
    _deg_body,
    out_type=jax.ShapeDtypeStruct((NC, N), jnp.float32),
    mesh=_mesh,
    compiler_params=pltpu.CompilerParams(use_tc_tiling_on_sc=False),
    scratch_types=[
        pltpu.VMEM((NCHUNK, CHUNK), jnp.int32),
        pltpu.VMEM((CHUNK + 16 - CHUNK % 16,), jnp.float32),
        pltpu.VMEM((N,), jnp.float32),
        pltpu.VMEM_SHARED((N,), jnp.float32),
    ],
)


CD = D // NC
EPW2 = E // NW
ECH = 100
NCH2 = EPW2 // ECH
NSTG = 4
HSTG = NCH2 // NSTG


def _edge_body(hp_hbm, src_hbm, dst_hbm, out_hbm,
               src_v, dst_v, buf0, buf1, buf2, acc, sem0, sem1, sem2):
    cid = lax.axis_index("c")
    sid = lax.axis_index("s")
    w = cid * NS + sid

    def zrow(r, _):
        for c in range(D // 16):
            buf0[r, pl.ds(c * 16, 16)] = jnp.zeros((16,), jnp.float32)
        return 0

    lax.fori_loop(0, ECH, zrow, 0)
    base = sid * ROWS_PER_TILE
    for k in range(6):
        pltpu.sync_copy(buf0, acc.at[pl.ds(base + k * ECH, ECH)])
    pltpu.sync_copy(buf0.at[pl.ds(0, 25)], acc.at[pl.ds(base + 600, 25)])
    plsc.subcore_barrier()

    bufs = ((buf0, sem0), (buf1, sem1), (buf2, sem2))
    NB = len(bufs)

    for q in range(NSTG):
        pltpu.sync_copy(src_hbm.at[w].at[pl.ds(q * HSTG, HSTG)], src_v)
        pltpu.sync_copy(dst_hbm.at[w].at[pl.ds(q * HSTG, HSTG)], dst_v)

        for b, (buf, sem) in enumerate(bufs):
            pltpu.async_copy(hp_hbm.at[src_v.at[b]], buf, sem)

        def body(jj, _):
            for b, (buf, sem) in enumerate(bufs):
                j = jj * NB + b
                pltpu.make_async_copy(hp_hbm.at[src_v.at[j]], buf,
                                      sem).wait()
                pltpu.sync_copy(buf, acc.at[dst_v.at[j]], add=True)

                @pl.when(j + NB < HSTG)
                def _():
                    pltpu.async_copy(hp_hbm.at[src_v.at[j + NB]], buf, sem)

            return 0

        lax.fori_loop(0, HSTG // NB, body, 0)

        j = HSTG - 1
        pltpu.make_async_copy(hp_hbm.at[src_v.at[j]], buf0, sem0).wait()
        pltpu.sync_copy(buf0, acc.at[dst_v.at[j]], add=True)

    plsc.subcore_barrier()

    ocid = out_hbm.at[cid]
    for k in range(6):
        r0 = base + k * ECH
        pltpu.sync_copy(acc.at[pl.ds(r0, ECH)], buf0)
        pltpu.sync_copy(buf0, ocid.at[pl.ds(r0, ECH)])
    r0 = base + 600
    pltpu.sync_copy(acc.at[pl.ds(r0, 25)], buf0.at[pl.ds(0, 25)])
    pltpu.sync_copy(buf0.at[pl.ds(0, 25)], ocid.at[pl.ds(r0, 25)])


_edge_call = pl.kernel(
    _edge_body,
    out_type=jax.ShapeDtypeStruct((NC, N, D), jnp.float32),
    mesh=_mesh,
    compiler_params=pltpu.CompilerParams(use_tc_tiling_on_sc=False),
    scratch_types=[
        pltpu.VMEM((HSTG, ECH), jnp.int32),
        pltpu.VMEM((HSTG, ECH), jnp.int32),
        pltpu.VMEM((ECH, D), jnp.float32),
        pltpu.VMEM((ECH, D), jnp.float32),
        pltpu.VMEM((ECH, D), jnp.float32),
        pltpu.VMEM_SHARED((N, D), jnp.float32),
        pltpu.SemaphoreType.DMA,
        pltpu.SemaphoreType.DMA,
        pltpu.SemaphoreType.DMA,
    ],
)


def _k0_body(degp_ref, x_ref, w1_ref, dinv_ref, hp_ref):
    deg = jnp.sum(degp_ref[...], axis=0) + 1.0
    dinv = lax.rsqrt(deg)
    dinv_ref[...] = dinv
    h = jnp.dot(x_ref[...], w1_ref[...], preferred_element_type=jnp.float32)
    hp_ref[...] = h * dinv[:, None]


def _k0(deg_parts, x, W1):
    return pl.pallas_call(
        _k0_body,
        out_shape=[
            jax.ShapeDtypeStruct((N,), jnp.float32),
            jax.ShapeDtypeStruct((N, D), jnp.float32),
        ],
    )(deg_parts, x, W1)


def _fuse_mid_body(pacc_ref, hp_ref, dinv_ref, b_ref, g_ref, be_ref,
                   w_ref, out_ref):
    dinv = dinv_ref[...]
    s = pacc_ref[0] + pacc_ref[1] + hp_ref[...]
    agg = s * dinv[:, None] + b_ref[...][None, :]
    mean = jnp.mean(agg, axis=0)
    var = jnp.mean(agg * agg, axis=0) - mean * mean
    xn = (agg - mean[None, :]) * lax.rsqrt(var + 1e-5)[None, :]
    h = jnp.maximum(xn * g_ref[...][None, :] + be_ref[...][None, :], 0.0)
    hw = jnp.dot(h, w_ref[...], preferred_element_type=jnp.float32)
    out_ref[...] = hw * dinv[:, None]


def _fuse_mid(pacc, hp, dinv, b, g, be, W):
    return pl.pallas_call(
        _fuse_mid_body,
        out_shape=jax.ShapeDtypeStruct((N, D), jnp.float32),
    )(pacc, hp, dinv, b, g, be, W)


def _fuse_tail_body(pacc_ref, hp_ref, dinv_ref, b_ref, gw1_ref, gb1_ref,
                    gw2_ref, gb2_ref, batch_ref, mw1_ref, mb1_ref, mw2_ref,
                    mb2_ref, out_ref):
    s = pacc_ref[0] + pacc_ref[1] + hp_ref[...]
    h3 = s * dinv_ref[...][:, None] + b_ref[...][None, :]
    gmid = jnp.maximum(
        jnp.dot(h3, gw1_ref[...], preferred_element_type=jnp.float32)
        + gb1_ref[...][None, :], 0.0)
    gate = (jnp.dot(gmid, gw2_ref[...], preferred_element_type=jnp.float32)
            + gb2_ref[...][None, :])[:, 0]
    oh = batch_ref[...][:, None] == lax.broadcasted_iota(jnp.int32, (1, G), 1)
    mx = jnp.max(jnp.where(oh, gate[:, None], -jnp.inf), axis=0)
    mx = jnp.where(jnp.isfinite(mx), mx, 0.0)
    ohf = oh.astype(jnp.float32)
    mxb = jnp.dot(ohf, mx[:, None], preferred_element_type=jnp.float32)[:, 0]
    e = jnp.exp(gate - mxb)
    den = jnp.sum(ohf * e[:, None], axis=0)
    ew = lax.dot_general(ohf, h3 * e[:, None], (((0,), (0,)), ((), ())),
                         preferred_element_type=jnp.float32)
    pooled = ew / (den + 1e-16)[:, None]
    m1 = jnp.maximum(
        jnp.dot(pooled, mw1_ref[...], preferred_element_type=jnp.float32)
        + mb1_ref[...][None, :], 0.0)
    out_ref[...] = (jnp.dot(m1, mw2_ref[...],
                            preferred_element_type=jnp.float32)
                    + mb2_ref[...][None, :])


def _fuse_tail(pacc, hp, dinv, b, gW1, gb1, gW2, gb2, batch,
               mW1, mb1, mW2, mb2):
    return pl.pallas_call(
        _fuse_tail_body,
        out_shape=jax.ShapeDtypeStruct((G, D), jnp.float32),
    )(pacc, hp, dinv, b, gW1, gb1, gW2, gb2, batch, mW1, mb1, mW2, mb2)


def kernel(x, edge_index, batch, W1, b1, W2, b2, W3, b3, g1, be1, g2, be2,
           gW1, gb1, gW2, gb2, mW1, mb1, mW2, mb2):
    src3 = edge_index[0].reshape(NW, NCH2, ECH)
    dst3 = edge_index[1].reshape(NW, NCH2, ECH)
    dst3d = edge_index[1].reshape(NW, NCHUNK, CHUNK)

    deg_parts = _deg_call(dst3d)
    dinv, hp1 = _k0(deg_parts, x, W1)

    pacc1 = _edge_call(hp1, src3, dst3)
    hp2 = _fuse_mid(pacc1, hp1, dinv, b1, g1, be1, W2)

    pacc2 = _edge_call(hp2, src3, dst3)
    hp3 = _fuse_mid(pacc2, hp2, dinv, b2, g2, be2, W3)

    pacc3 = _edge_call(hp3, src3, dst3)
    return _fuse_tail(pacc3, hp3, dinv, b3, gW1, gb1, gW2, gb2, batch,
                      mW1, mb1, mW2, mb2)


# speedup vs baseline: 1.0308x; 1.0006x over previous
import jax
import jax.numpy as jnp
from jax import lax
from jax.experimental import pallas as pl
from jax.experimental.pallas import tpu as pltpu
from jax.experimental.pallas import tpu_sc as plsc

N = 10000
E = 320000
D = 128
G = 16

NC = 2
NS = 16
NW = NC * NS
EPW = E // NW
CHUNK = 125
NCHUNK = EPW // CHUNK
ROWS_PER_TILE = N // NS

_mesh = plsc.VectorSubcoreMesh(core_axis_name="c", subcore_axis_name="s")


def _deg_body(dst_hbm, out_hbm, dst_v, ones_v, zbuf, acc):
    cid = lax.axis_index("c")
    sid = lax.axis_index("s")
    w = cid * NS + sid

    def zinit(i, _):
        zbuf[pl.ds(i * 16, 16)] = jnp.zeros((16,), jnp.float32)
        return 0

    lax.fori_loop(0, N // 16, zinit, 0)

    @pl.when(sid == 0)
    def _():
        pltpu.sync_copy(zbuf, acc)

    def oinit(i, _):
        ones_v[pl.ds(i * 16, 16)] = jnp.ones((16,), jnp.float32)
        return 0

    lax.fori_loop(0, CHUNK // 16 + 1, oinit, 0)
    plsc.subcore_barrier()

    pltpu.sync_copy(dst_hbm.at[w], dst_v)

    def body(j, _):
        pltpu.sync_copy(ones_v.at[pl.ds(0, CHUNK)], acc.at[dst_v.at[j]],
                        add=True)
        return 0

    lax.fori_loop(0, NCHUNK, body, 0)
    plsc.subcore_barrier()

    @pl.when(sid == 0)
    def _():
        pltpu.sync_copy(acc, out_hbm.at[cid])


_deg_call = pl.kernel(
    _deg_body,
    out_type=jax.ShapeDtypeStruct((NC, N), jnp.float32),
    mesh=_mesh,
    compiler_params=pltpu.CompilerParams(use_tc_tiling_on_sc=False),
    scratch_types=[
        pltpu.VMEM((NCHUNK, CHUNK), jnp.int32),
        pltpu.VMEM((CHUNK + 16 - CHUNK % 16,), jnp.float32),
        pltpu.VMEM((N,), jnp.float32),
        pltpu.VMEM_SHARED((N,), jnp.float32),
    ],
)


EPW2 = E // NW
ECH = 100
NCH2 = EPW2 // ECH
NSTG = 4
HSTG = NCH2 // NSTG


def _edge_body(hp_hbm, src_hbm, dst_hbm, out_hbm,
               src_v, dst_v, buf0, buf1, buf2, acc, sem0, sem1, sem2):
    cid = lax.axis_index("c")
    sid = lax.axis_index("s")
    w = cid * NS + sid

    def zrow(r, _):
        for c in range(D // 16):
            buf0[r, pl.ds(c * 16, 16)] = jnp.zeros((16,), jnp.float32)
        return 0

    lax.fori_loop(0, ECH, zrow, 0)
    base = sid * ROWS_PER_TILE
    for k in range(6):
        pltpu.sync_copy(buf0, acc.at[pl.ds(base + k * ECH, ECH)])
    pltpu.sync_copy(buf0.at[pl.ds(0, 25)], acc.at[pl.ds(base + 600, 25)])
    plsc.subcore_barrier()

    bufs = ((buf0, sem0), (buf1, sem1), (buf2, sem2))
    NB = len(bufs)

    for q in range(NSTG):
        pltpu.sync_copy(src_hbm.at[w].at[pl.ds(q * HSTG, HSTG)], src_v)
        pltpu.sync_copy(dst_hbm.at[w].at[pl.ds(q * HSTG, HSTG)], dst_v)

        for b, (buf, sem) in enumerate(bufs):
            pltpu.async_copy(hp_hbm.at[src_v.at[b]], buf, sem)

        def body(jj, _):
            for b, (buf, sem) in enumerate(bufs):
                j = jj * NB + b
                pltpu.make_async_copy(hp_hbm.at[src_v.at[j]], buf,
                                      sem).wait()
                pltpu.sync_copy(buf, acc.at[dst_v.at[j]], add=True)

                @pl.when(j + NB < HSTG)
                def _():
                    pltpu.async_copy(hp_hbm.at[src_v.at[j + NB]], buf, sem)

            return 0

        lax.fori_loop(0, HSTG // NB, body, 0)

        j = HSTG - 1
        pltpu.make_async_copy(hp_hbm.at[src_v.at[j]], buf0, sem0).wait()
        pltpu.sync_copy(buf0, acc.at[dst_v.at[j]], add=True)

    plsc.subcore_barrier()

    ocid = out_hbm.at[cid]
    for k in range(6):
        r0 = base + k * ECH
        pltpu.sync_copy(acc.at[pl.ds(r0, ECH)], buf0)
        pltpu.sync_copy(buf0, ocid.at[pl.ds(r0, ECH)])
    r0 = base + 600
    pltpu.sync_copy(acc.at[pl.ds(r0, 25)], buf0.at[pl.ds(0, 25)])
    pltpu.sync_copy(buf0.at[pl.ds(0, 25)], ocid.at[pl.ds(r0, 25)])


_edge_call = pl.kernel(
    _edge_body,
    out_type=jax.ShapeDtypeStruct((NC, N, D), jnp.float32),
    mesh=_mesh,
    compiler_params=pltpu.CompilerParams(use_tc_tiling_on_sc=False),
    scratch_types=[
        pltpu.VMEM((HSTG, ECH), jnp.int32),
        pltpu.VMEM((HSTG, ECH), jnp.int32),
        pltpu.VMEM((ECH, D), jnp.float32),
        pltpu.VMEM((ECH, D), jnp.float32),
        pltpu.VMEM((ECH, D), jnp.float32),
        pltpu.VMEM_SHARED((N, D), jnp.float32),
        pltpu.SemaphoreType.DMA,
        pltpu.SemaphoreType.DMA,
        pltpu.SemaphoreType.DMA,
    ],
)


def _k0_body(degp_ref, x_ref, w1_ref, dinv_ref, hp_ref):
    deg = jnp.sum(degp_ref[...], axis=0) + 1.0
    dinv = lax.rsqrt(deg)
    dinv_ref[...] = dinv
    h = jnp.dot(x_ref[...], w1_ref[...], preferred_element_type=jnp.float32)
    hp_ref[...] = h * dinv[:, None]


def _k0(deg_parts, x, W1):
    return pl.pallas_call(
        _k0_body,
        out_shape=[
            jax.ShapeDtypeStruct((N,), jnp.float32),
            jax.ShapeDtypeStruct((N, D), jnp.float32),
        ],
    )(deg_parts, x, W1)


def _fuse_mid_body(pacc_ref, hp_ref, dinv_ref, b_ref, g_ref, be_ref,
                   w_ref, out_ref):
    dinv = dinv_ref[...]
    s = pacc_ref[0] + pacc_ref[1] + hp_ref[...]
    agg = s * dinv[:, None] + b_ref[...][None, :]
    mean = jnp.mean(agg, axis=0)
    var = jnp.mean(agg * agg, axis=0) - mean * mean
    xn = (agg - mean[None, :]) * lax.rsqrt(var + 1e-5)[None, :]
    h = jnp.maximum(xn * g_ref[...][None, :] + be_ref[...][None, :], 0.0)
    hw = jnp.dot(h, w_ref[...], preferred_element_type=jnp.float32)
    out_ref[...] = hw * dinv[:, None]


def _fuse_mid(pacc, hp, dinv, b, g, be, W):
    return pl.pallas_call(
        _fuse_mid_body,
        out_shape=jax.ShapeDtypeStruct((N, D), jnp.float32),
    )(pacc, hp, dinv, b, g, be, W)


def _fuse_tail_body(pacc_ref, hp_ref, dinv_ref, b_ref, gw1_ref, gb1_ref,
                    gw2_ref, gb2_ref, batch_ref, mw1_ref, mb1_ref, mw2_ref,
                    mb2_ref, out_ref):
    s = pacc_ref[0] + pacc_ref[1] + hp_ref[...]
    h3 = s * dinv_ref[...][:, None] + b_ref[...][None, :]
    gmid = jnp.maximum(
        jnp.dot(h3, gw1_ref[...], preferred_element_type=jnp.float32)
        + gb1_ref[...][None, :], 0.0)
    gate = (jnp.dot(gmid, gw2_ref[...], preferred_element_type=jnp.float32)
            + gb2_ref[...][None, :])[:, 0]
    oh = batch_ref[...][:, None] == lax.broadcasted_iota(jnp.int32, (1, G), 1)
    mx = jnp.max(jnp.where(oh, gate[:, None], -jnp.inf), axis=0)
    mx = jnp.where(jnp.isfinite(mx), mx, 0.0)
    ohf = oh.astype(jnp.float32)
    mxb = jnp.dot(ohf, mx[:, None], preferred_element_type=jnp.float32)[:, 0]
    e = jnp.exp(gate - mxb)
    den = jnp.sum(ohf * e[:, None], axis=0)
    ew = lax.dot_general(ohf, h3 * e[:, None], (((0,), (0,)), ((), ())),
                         preferred_element_type=jnp.float32)
    pooled = ew / (den + 1e-16)[:, None]
    m1 = jnp.maximum(
        jnp.dot(pooled, mw1_ref[...], preferred_element_type=jnp.float32)
        + mb1_ref[...][None, :], 0.0)
    out_ref[...] = (jnp.dot(m1, mw2_ref[...],
                            preferred_element_type=jnp.float32)
                    + mb2_ref[...][None, :])


def _fuse_tail(pacc, hp, dinv, b, gW1, gb1, gW2, gb2, batch,
               mW1, mb1, mW2, mb2):
    return pl.pallas_call(
        _fuse_tail_body,
        out_shape=jax.ShapeDtypeStruct((G, D), jnp.float32),
    )(pacc, hp, dinv, b, gW1, gb1, gW2, gb2, batch, mW1, mb1, mW2, mb2)


def kernel(x, edge_index, batch, W1, b1, W2, b2, W3, b3, g1, be1, g2, be2,
           gW1, gb1, gW2, gb2, mW1, mb1, mW2, mb2):
    src3 = edge_index[0].reshape(NW, NCH2, ECH)
    dst3 = edge_index[1].reshape(NW, NCH2, ECH)
    dst3d = edge_index[1].reshape(NW, NCHUNK, CHUNK)

    deg_parts = _deg_call(dst3d)
    dinv, hp1 = _k0(deg_parts, x, W1)

    pacc1 = _edge_call(hp1, src3, dst3)
    hp2 = _fuse_mid(pacc1, hp1, dinv, b1, g1, be1, W2)

    pacc2 = _edge_call(hp2, src3, dst3)
    hp3 = _fuse_mid(pacc2, hp2, dinv, b2, g2, be2, W3)

    pacc3 = _edge_call(hp3, src3, dst3)
    return _fuse_tail(pacc3, hp3, dinv, b3, gW1, gb1, gW2, gb2, batch,
                      mW1, mb1, mW2, mb2)
